# baseline (speedup 1.0000x reference)
import jax
import jax.numpy as jnp
from jax.experimental import pallas as pl
from jax.experimental.pallas import tpu as pltpu

_NBUF = 4


def _fused_body(
    a_hbm, h_ref, wa_ref, wb_ref, o_ref, abuf, hg_ref, p_ref, q_ref, s_ref, sems
):
    i = pl.program_id(0)
    ni = pl.num_programs(0) - (p_ref.shape[0] // o_ref.shape[0])
    bi = abuf.shape[1]
    d = wa_ref.shape[0]

    def stripe_copy(stripe, slot):
        return pltpu.make_async_copy(
            a_hbm.at[pl.ds(stripe * bi, bi), :],
            abuf.at[slot],
            sems.at[slot],
        )

    @pl.when(i == 0)
    def _init():
        s_ref[...] = jnp.zeros_like(s_ref)
        hg_ref[:, :d] = h_ref[...]
        hg_ref[:, d:] = jnp.ones_like(hg_ref[:, d:])
        for b in range(1, _NBUF):
            stripe_copy(b, b).start()

    @pl.when(i < ni)
    def _stripe():
        slot = jax.lax.rem(i, _NBUF)
        @pl.when(i == 0)
        def _first():
            stripe_copy(0, 0).start()

        stripe_copy(i, slot).wait()
        r0 = i * bi
        a = abuf[slot]
        u_aug = jnp.dot(a, hg_ref[...], preferred_element_type=jnp.float32)
        s_ref[...] += jnp.sum(u_aug[:, d])[None, None]
        q_ref[pl.ds(r0, bi), :] = jnp.dot(
            u_aug[:, :d], wb_ref[...], preferred_element_type=jnp.float32
        )
        p_ref[pl.ds(r0, bi), :] = jnp.dot(
            h_ref[pl.ds(r0, bi), :], wa_ref[...], preferred_element_type=jnp.float32
        )

        @pl.when(i + _NBUF < ni)
        def _prefetch():
            stripe_copy(i + _NBUF, slot).start()

    @pl.when(i >= ni)
    def _finale():
        ce = o_ref.shape[0]
        c0 = (i - ni) * ce
        inv = 1.0 / s_ref[0, 0]
        o_ref[...] = jnp.maximum(
            p_ref[pl.ds(c0, ce), :] + q_ref[pl.ds(c0, ce), :] * inv, 0.0
        )


def kernel(h, A, W):
    n, d = h.shape
    bi = 200
    ni = n // bi
    k = 5
    ce = n // k
    daug = d + 8

    wt = W.T
    wa = wt[:d]
    wb = wt[d:]

    out = pl.pallas_call(
        _fused_body,
        grid=(ni + k,),
        in_specs=[
            pl.BlockSpec(memory_space=pltpu.MemorySpace.HBM),
            pl.BlockSpec((n, d), lambda i: (0, 0)),
            pl.BlockSpec((d, d), lambda i: (0, 0)),
            pl.BlockSpec((d, d), lambda i: (0, 0)),
        ],
        out_specs=pl.BlockSpec((ce, d), lambda i: (jnp.maximum(i - ni, 0), 0)),
        out_shape=jax.ShapeDtypeStruct((n, d), jnp.float32),
        scratch_shapes=[
            pltpu.VMEM((_NBUF, bi, n), jnp.float32),
            pltpu.VMEM((n, daug), jnp.float32),
            pltpu.VMEM((n, d), jnp.float32),
            pltpu.VMEM((n, d), jnp.float32),
            pltpu.VMEM((1, 1), jnp.float32),
            pltpu.SemaphoreType.DMA((_NBUF,)),
        ],
        compiler_params=pltpu.CompilerParams(
            dimension_semantics=("arbitrary",),
        ),
    )(A, h, wa, wb)
    return out


# speedup vs baseline: 1.0173x; 1.0173x over previous
import jax
import jax.numpy as jnp
from jax.experimental import pallas as pl
from jax.experimental.pallas import tpu as pltpu

_NBUF = 3


def _fused_body(
    a_hbm, h_ref, wa_ref, wb_ref, o_ref, abuf, hg_ref, p_ref, q_ref, s_ref, sems
):
    i = pl.program_id(0)
    ni = pl.num_programs(0) - (p_ref.shape[0] // o_ref.shape[0])
    bi = abuf.shape[1]
    d = wa_ref.shape[0]

    def stripe_copies(stripe, slot):
        c0 = 4992
        return (
            pltpu.make_async_copy(
                a_hbm.at[pl.ds(stripe * bi, bi), pl.ds(0, c0)],
                abuf.at[slot, :, pl.ds(0, c0)],
                sems.at[slot, 0],
            ),
            pltpu.make_async_copy(
                a_hbm.at[pl.ds(stripe * bi, bi), pl.ds(c0, 10000 - c0)],
                abuf.at[slot, :, pl.ds(c0, 10000 - c0)],
                sems.at[slot, 1],
            ),
        )

    def start_stripe(stripe, slot):
        for c in stripe_copies(stripe, slot):
            c.start()

    def wait_stripe(stripe, slot):
        for c in stripe_copies(stripe, slot):
            c.wait()

    @pl.when(i == 0)
    def _init():
        s_ref[...] = jnp.zeros_like(s_ref)
        hg_ref[:, :d] = h_ref[...]
        hg_ref[:, d:] = jnp.ones_like(hg_ref[:, d:])
        for b in range(1, _NBUF):
            start_stripe(b, b)

    @pl.when(i < ni)
    def _stripe():
        slot = jax.lax.rem(i, _NBUF)
        @pl.when(i == 0)
        def _first():
            start_stripe(0, 0)

        wait_stripe(i, slot)
        r0 = i * bi
        a = abuf[slot]
        u_aug = jnp.dot(a, hg_ref[...], preferred_element_type=jnp.float32)
        s_ref[...] += jnp.sum(u_aug[:, d])[None, None]
        q_ref[pl.ds(r0, bi), :] = jnp.dot(
            u_aug[:, :d], wb_ref[...], preferred_element_type=jnp.float32
        )
        p_ref[pl.ds(r0, bi), :] = jnp.dot(
            h_ref[pl.ds(r0, bi), :], wa_ref[...], preferred_element_type=jnp.float32
        )

        @pl.when(i + _NBUF < ni)
        def _prefetch():
            start_stripe(i + _NBUF, slot)

    @pl.when(i >= ni)
    def _finale():
        ce = o_ref.shape[0]
        c0 = (i - ni) * ce
        inv = 1.0 / s_ref[0, 0]
        o_ref[...] = jnp.maximum(
            p_ref[pl.ds(c0, ce), :] + q_ref[pl.ds(c0, ce), :] * inv, 0.0
        )


def kernel(h, A, W):
    n, d = h.shape
    bi = 200
    ni = n // bi
    k = 5
    ce = n // k
    daug = d + 8

    wt = W.T
    wa = wt[:d]
    wb = wt[d:]

    out = pl.pallas_call(
        _fused_body,
        grid=(ni + k,),
        in_specs=[
            pl.BlockSpec(memory_space=pltpu.MemorySpace.HBM),
            pl.BlockSpec((n, d), lambda i: (0, 0)),
            pl.BlockSpec((d, d), lambda i: (0, 0)),
            pl.BlockSpec((d, d), lambda i: (0, 0)),
        ],
        out_specs=pl.BlockSpec((ce, d), lambda i: (jnp.maximum(i - ni, 0), 0)),
        out_shape=jax.ShapeDtypeStruct((n, d), jnp.float32),
        scratch_shapes=[
            pltpu.VMEM((_NBUF, bi, n), jnp.float32),
            pltpu.VMEM((n, daug), jnp.float32),
            pltpu.VMEM((n, d), jnp.float32),
            pltpu.VMEM((n, d), jnp.float32),
            pltpu.VMEM((1, 1), jnp.float32),
            pltpu.SemaphoreType.DMA((_NBUF, 2)),
        ],
        compiler_params=pltpu.CompilerParams(
            dimension_semantics=("arbitrary",),
        ),
    )(A, h, wa, wb)
    return out


# speedup vs baseline: 1.0366x; 1.0190x over previous
import jax
import jax.numpy as jnp
from jax.experimental import pallas as pl
from jax.experimental.pallas import tpu as pltpu

_NBUF = 3


def _fused_body(
    a_hbm, h_hbm, wa_ref, wb_ref, o_ref,
    abuf, hv_ref, hg_ref, p_ref, q_ref, s_ref, sems,
):
    i = pl.program_id(0)
    ni = pl.num_programs(0) - (p_ref.shape[0] // o_ref.shape[0])
    bi = abuf.shape[1]
    d = wa_ref.shape[0]

    def stripe_copy(stripe, slot):
        return pltpu.make_async_copy(
            a_hbm.at[pl.ds(stripe * bi, bi), :],
            abuf.at[slot],
            sems.at[slot],
        )

    def h_copy():
        return pltpu.make_async_copy(h_hbm, hv_ref, sems.at[_NBUF])

    @pl.when(i == 0)
    def _init():
        h_copy().start()
        for b in range(_NBUF):
            stripe_copy(b, b).start()
        s_ref[...] = jnp.zeros_like(s_ref)
        h_copy().wait()
        hg_ref[:, :d] = hv_ref[...]
        hg_ref[:, d:] = jnp.ones_like(hg_ref[:, d:])

    @pl.when(i < ni)
    def _stripe():
        slot = jax.lax.rem(i, _NBUF)
        stripe_copy(i, slot).wait()
        r0 = i * bi
        a = abuf[slot]
        u_aug = jnp.dot(a, hg_ref[...], preferred_element_type=jnp.float32)
        s_ref[...] += jnp.sum(u_aug[:, d])[None, None]
        q_ref[pl.ds(r0, bi), :] = jnp.dot(
            u_aug[:, :d], wb_ref[...], preferred_element_type=jnp.float32
        )
        p_ref[pl.ds(r0, bi), :] = jnp.dot(
            hv_ref[pl.ds(r0, bi), :], wa_ref[...], preferred_element_type=jnp.float32
        )

        @pl.when(i + _NBUF < ni)
        def _prefetch():
            stripe_copy(i + _NBUF, slot).start()

    @pl.when(i >= ni)
    def _finale():
        ce = o_ref.shape[0]
        c0 = (i - ni) * ce
        inv = 1.0 / s_ref[0, 0]
        o_ref[...] = jnp.maximum(
            p_ref[pl.ds(c0, ce), :] + q_ref[pl.ds(c0, ce), :] * inv, 0.0
        )


def kernel(h, A, W):
    n, d = h.shape
    bi = 200
    ni = n // bi
    k = 5
    ce = n // k
    daug = d + 8

    wt = W.T
    wa = wt[:d]
    wb = wt[d:]

    out = pl.pallas_call(
        _fused_body,
        grid=(ni + k,),
        in_specs=[
            pl.BlockSpec(memory_space=pltpu.MemorySpace.HBM),
            pl.BlockSpec(memory_space=pltpu.MemorySpace.HBM),
            pl.BlockSpec((d, d), lambda i: (0, 0)),
            pl.BlockSpec((d, d), lambda i: (0, 0)),
        ],
        out_specs=pl.BlockSpec((ce, d), lambda i: (jnp.maximum(i - ni, 0), 0)),
        out_shape=jax.ShapeDtypeStruct((n, d), jnp.float32),
        scratch_shapes=[
            pltpu.VMEM((_NBUF, bi, n), jnp.float32),
            pltpu.VMEM((n, d), jnp.float32),
            pltpu.VMEM((n, daug), jnp.float32),
            pltpu.VMEM((n, d), jnp.float32),
            pltpu.VMEM((n, d), jnp.float32),
            pltpu.VMEM((1, 1), jnp.float32),
            pltpu.SemaphoreType.DMA((_NBUF + 1,)),
        ],
        compiler_params=pltpu.CompilerParams(
            dimension_semantics=("arbitrary",),
        ),
    )(A, h, wa, wb)
    return out
